# Initial kernel scaffold; baseline (speedup 1.0000x reference)
import jax
import jax.numpy as jnp
from jax.experimental import pallas as pl


def kernel(h, edge_index, W, b):
    raise NotImplementedError("write your pallas kernel here")


# speedup vs baseline: 3.1627x; 3.1627x over previous
import functools

import jax
import jax.numpy as jnp
from jax import lax
from jax.experimental import pallas as pl
from jax.experimental.pallas import tpu as pltpu
from jax.experimental.pallas import tpu_sc as plsc

_NUM_CORES = 2
_NUM_SUBCORES = 16
_NW = _NUM_CORES * _NUM_SUBCORES
_LANES = 16

_N = 10000
_E = 320000
_D = 128
_DT = 2 * _D

_C = 80
_NCHUNKS = _E // _C


def _tables_kernel(h_ref, a_ref, b_ref):
    hh = h_ref[...]
    sh = jnp.sin(hh)
    ch = jnp.cos(hh)
    a_ref[...] = jnp.concatenate([sh, ch], axis=1)
    b_ref[...] = jnp.concatenate([ch, -sh], axis=1)


def _make_tables(h):
    rows = 1000
    grid = _N // rows
    return pl.pallas_call(
        _tables_kernel,
        grid=(grid,),
        in_specs=[
            pl.BlockSpec((rows, _D), lambda i: (i, 0)),
        ],
        out_specs=[
            pl.BlockSpec((rows, _DT), lambda i: (i, 0)),
            pl.BlockSpec((rows, _DT), lambda i: (i, 0)),
        ],
        out_shape=[
            jax.ShapeDtypeStruct((_N, _DT), jnp.float32),
            jax.ShapeDtypeStruct((_N, _DT), jnp.float32),
        ],
    )(h)


def _round_bf16(t):
    i = plsc.bitcast(t, jnp.int32)
    lsb = lax.shift_right_logical(i, 16) & 1
    i = (i + 0x7FFF + lsb) & jnp.int32(-65536)
    return plsc.bitcast(i, jnp.float32)


def _sc_kernel(a_hbm, b_hbm, src_hbm, dst_hbm, w_hbm, out_hbm,
               sidx, didx, arows, brows, scbuf, pbuf, stage, wbuf,
               sem_a, sem_b):
    wid = lax.axis_index("s") * _NUM_CORES + lax.axis_index("c")
    nchunks = _NCHUNKS // _NW
    iota = lax.iota(jnp.int32, _LANES)

    pltpu.sync_copy(w_hbm, wbuf)
    wv = [wbuf[pl.ds(j * _LANES, _LANES)] for j in range(_D // _LANES)]

    def chunk_body(t, _):
        cid = wid + t * _NW
        off = cid * _C
        pltpu.sync_copy(src_hbm.at[pl.ds(off, _C)], sidx)
        pltpu.sync_copy(dst_hbm.at[pl.ds(off, _C)], didx)
        cp_a = pltpu.async_copy(a_hbm.at[sidx], arows, sem_a)
        cp_b = pltpu.async_copy(b_hbm.at[didx], brows, sem_b)
        cp_a.wait()
        cp_b.wait()

        def blk_body(blk, _):
            for u in range(_LANES):
                e = blk * _LANES + u
                acc = jnp.zeros((_LANES,), jnp.float32)
                for j in range(_D // _LANES):
                    sin_s = arows[e, pl.ds(j * _LANES, _LANES)]
                    cos_s = arows[e, pl.ds(_D + j * _LANES, _LANES)]
                    cos_o = brows[e, pl.ds(j * _LANES, _LANES)]
                    nsin_o = brows[e, pl.ds(_D + j * _LANES, _LANES)]
                    sin_d = sin_s * cos_o + cos_s * nsin_o
                    acc = acc + _round_bf16(sin_d) * wv[j]
                stage[pl.ds(u * _LANES, _LANES)] = acc
            score = jnp.zeros((_LANES,), jnp.float32)
            col = iota * _LANES
            for c in range(_LANES):
                score = score + plsc.load_gather(stage, [col + c])
            scbuf[pl.ds(blk * _LANES, _LANES)] = score
            return _

        lax.fori_loop(0, _C // _LANES, blk_body, None)

        for w0 in (0, 32, 48):
            idx_e = w0 + 2 * iota
            idx_o = idx_e + 1
            x0 = plsc.load_gather(scbuf, [idx_e])
            x1 = plsc.load_gather(scbuf, [idx_o])
            m = jnp.maximum(x0, x1)
            e0 = jnp.exp(x0 - m)
            e1 = jnp.exp(x1 - m)
            s = e0 + e1
            plsc.store_scatter(pbuf, [idx_e], e0 / s)
            plsc.store_scatter(pbuf, [idx_o], e1 / s)

        pltpu.sync_copy(pbuf, out_hbm.at[pl.ds(off, _C)])
        return _

    lax.fori_loop(0, nchunks, chunk_body, None)


@functools.partial(
    pl.kernel,
    mesh=plsc.VectorSubcoreMesh(core_axis_name="c", subcore_axis_name="s"),
    compiler_params=pltpu.CompilerParams(needs_layout_passes=False),
    out_type=jax.ShapeDtypeStruct((_E,), jnp.float32),
    scratch_types=[
        pltpu.VMEM((_C,), jnp.int32),
        pltpu.VMEM((_C,), jnp.int32),
        pltpu.VMEM((_C, _DT), jnp.float32),
        pltpu.VMEM((_C, _DT), jnp.float32),
        pltpu.VMEM((_C,), jnp.float32),
        pltpu.VMEM((_C,), jnp.float32),
        pltpu.VMEM((_LANES * _LANES,), jnp.float32),
        pltpu.VMEM((_D,), jnp.float32),
        pltpu.SemaphoreType.DMA,
        pltpu.SemaphoreType.DMA,
    ],
)
def _sc_scores(a_hbm, b_hbm, src_hbm, dst_hbm, w_hbm, out_hbm,
               sidx, didx, arows, brows, scbuf, pbuf, stage, wbuf,
               sem_a, sem_b):
    _sc_kernel(a_hbm, b_hbm, src_hbm, dst_hbm, w_hbm, out_hbm,
               sidx, didx, arows, brows, scbuf, pbuf, stage, wbuf,
               sem_a, sem_b)


def kernel(h, edge_index, W, b):
    del b
    A, B = _make_tables(h)
    wi = lax.bitcast_convert_type(W.reshape(-1), jnp.int32)
    wlsb = lax.shift_right_logical(wi, 16) & 1
    wi = (wi + 0x7FFF + wlsb) & jnp.int32(-65536)
    w_bf = lax.bitcast_convert_type(wi, jnp.float32)
    src = edge_index[0]
    dst = edge_index[1]
    probs = _sc_scores(A, B, src, dst, w_bf)
    score = probs.reshape(-1, 1)
    return (score, score > 0.5)


# speedup vs baseline: 4.1627x; 1.3162x over previous
import functools

import jax
import jax.numpy as jnp
from jax import lax
from jax.experimental import pallas as pl
from jax.experimental.pallas import tpu as pltpu
from jax.experimental.pallas import tpu_sc as plsc

_NUM_CORES = 2
_NUM_SUBCORES = 16
_NW = _NUM_CORES * _NUM_SUBCORES
_LANES = 16

_N = 10000
_E = 320000
_D = 128
_DT = 2 * _D

_C = 80
_NCHUNKS = _E // _C
_PER_W = _NCHUNKS // _NW


def _tables_kernel(h_ref, t_ref):
    hh = h_ref[...]
    t_ref[...] = jnp.concatenate([jnp.sin(hh), jnp.cos(hh)], axis=1)


def _make_table(h):
    rows = 1000
    grid = _N // rows
    return pl.pallas_call(
        _tables_kernel,
        grid=(grid,),
        in_specs=[pl.BlockSpec((rows, _D), lambda i: (i, 0))],
        out_specs=pl.BlockSpec((rows, _DT), lambda i: (i, 0)),
        out_shape=jax.ShapeDtypeStruct((_N, _DT), jnp.float32),
    )(h)


def _round_bf16(t):
    i = plsc.bitcast(t, jnp.int32)
    lsb = lax.shift_right_logical(i, 16) & 1
    i = (i + 0x7FFF + lsb) & jnp.int32(-65536)
    return plsc.bitcast(i, jnp.float32)


def _sc_kernel(t_hbm, src_hbm, dst_hbm, w_hbm, out_hbm,
               sidx0, didx0, sidx1, didx1, arows0, brows0, arows1, brows1,
               scbuf, pbuf, stage, wbuf, sem_a0, sem_b0, sem_a1, sem_b1):
    wid = lax.axis_index("s") * _NUM_CORES + lax.axis_index("c")
    iota = lax.iota(jnp.int32, _LANES)

    pltpu.sync_copy(w_hbm, wbuf)
    wv = [wbuf[pl.ds(j * _LANES, _LANES)] for j in range(_D // _LANES)]

    def load_idx(t, sidx, didx):
        off = (wid + t * _NW) * _C
        pltpu.sync_copy(src_hbm.at[pl.ds(off, _C)], sidx)
        pltpu.sync_copy(dst_hbm.at[pl.ds(off, _C)], didx)

    def fire(sidx, didx, arows, brows, sa, sb):
        pltpu.async_copy(t_hbm.at[sidx], arows, sa)
        pltpu.async_copy(t_hbm.at[didx], brows, sb)

    def drain(sidx, didx, arows, brows, sa, sb):
        pltpu.make_async_copy(t_hbm.at[sidx], arows, sa).wait()
        pltpu.make_async_copy(t_hbm.at[didx], brows, sb).wait()

    def compute(t, arows, brows):
        off = (wid + t * _NW) * _C

        def blk_body(blk, _):
            for u in range(_LANES):
                e = blk * _LANES + u
                acc = jnp.zeros((_LANES,), jnp.float32)
                for j in range(_D // _LANES):
                    sin_s = arows[e, pl.ds(j * _LANES, _LANES)]
                    cos_s = arows[e, pl.ds(_D + j * _LANES, _LANES)]
                    sin_o = brows[e, pl.ds(j * _LANES, _LANES)]
                    cos_o = brows[e, pl.ds(_D + j * _LANES, _LANES)]
                    sin_d = sin_s * cos_o - cos_s * sin_o
                    acc = acc + _round_bf16(sin_d) * wv[j]
                stage[pl.ds(u * _LANES, _LANES)] = acc
            score = jnp.zeros((_LANES,), jnp.float32)
            col = iota * _LANES
            for c in range(_LANES):
                score = score + plsc.load_gather(stage, [col + c])
            scbuf[pl.ds(blk * _LANES, _LANES)] = score
            return _

        lax.fori_loop(0, _C // _LANES, blk_body, None)

        for w0 in (0, 32, 48):
            idx_e = w0 + 2 * iota
            idx_o = idx_e + 1
            x0 = plsc.load_gather(scbuf, [idx_e])
            x1 = plsc.load_gather(scbuf, [idx_o])
            m = jnp.maximum(x0, x1)
            e0 = jnp.exp(x0 - m)
            e1 = jnp.exp(x1 - m)
            s = e0 + e1
            plsc.store_scatter(pbuf, [idx_e], e0 / s)
            plsc.store_scatter(pbuf, [idx_o], e1 / s)

        pltpu.sync_copy(pbuf, out_hbm.at[pl.ds(off, _C)])

    load_idx(0, sidx0, didx0)
    fire(sidx0, didx0, arows0, brows0, sem_a0, sem_b0)

    def body(g, _):
        t0 = 2 * g
        load_idx(t0 + 1, sidx1, didx1)
        fire(sidx1, didx1, arows1, brows1, sem_a1, sem_b1)
        drain(sidx0, didx0, arows0, brows0, sem_a0, sem_b0)
        compute(t0, arows0, brows0)
        load_idx(t0 + 2, sidx0, didx0)
        fire(sidx0, didx0, arows0, brows0, sem_a0, sem_b0)
        drain(sidx1, didx1, arows1, brows1, sem_a1, sem_b1)
        compute(t0 + 1, arows1, brows1)
        return _

    lax.fori_loop(0, (_PER_W - 1) // 2, body, None)
    drain(sidx0, didx0, arows0, brows0, sem_a0, sem_b0)
    compute(_PER_W - 1, arows0, brows0)


@functools.partial(
    pl.kernel,
    mesh=plsc.VectorSubcoreMesh(core_axis_name="c", subcore_axis_name="s"),
    compiler_params=pltpu.CompilerParams(needs_layout_passes=False),
    out_type=jax.ShapeDtypeStruct((_E,), jnp.float32),
    scratch_types=[
        pltpu.VMEM((_C,), jnp.int32),
        pltpu.VMEM((_C,), jnp.int32),
        pltpu.VMEM((_C,), jnp.int32),
        pltpu.VMEM((_C,), jnp.int32),
        pltpu.VMEM((_C, _DT), jnp.float32),
        pltpu.VMEM((_C, _DT), jnp.float32),
        pltpu.VMEM((_C, _DT), jnp.float32),
        pltpu.VMEM((_C, _DT), jnp.float32),
        pltpu.VMEM((_C,), jnp.float32),
        pltpu.VMEM((_C,), jnp.float32),
        pltpu.VMEM((_LANES * _LANES,), jnp.float32),
        pltpu.VMEM((_D,), jnp.float32),
        pltpu.SemaphoreType.DMA,
        pltpu.SemaphoreType.DMA,
        pltpu.SemaphoreType.DMA,
        pltpu.SemaphoreType.DMA,
    ],
)
def _sc_scores(t_hbm, src_hbm, dst_hbm, w_hbm, out_hbm,
               sidx0, didx0, sidx1, didx1, arows0, brows0, arows1, brows1,
               scbuf, pbuf, stage, wbuf, sem_a0, sem_b0, sem_a1, sem_b1):
    _sc_kernel(t_hbm, src_hbm, dst_hbm, w_hbm, out_hbm,
               sidx0, didx0, sidx1, didx1, arows0, brows0, arows1, brows1,
               scbuf, pbuf, stage, wbuf, sem_a0, sem_b0, sem_a1, sem_b1)


def kernel(h, edge_index, W, b):
    del b
    T = _make_table(h)
    wi = lax.bitcast_convert_type(W.reshape(-1), jnp.int32)
    wlsb = lax.shift_right_logical(wi, 16) & 1
    wi = (wi + 0x7FFF + wlsb) & jnp.int32(-65536)
    w_bf = lax.bitcast_convert_type(wi, jnp.float32)
    src = edge_index[0]
    dst = edge_index[1]
    probs = _sc_scores(T, src, dst, w_bf)
    score = probs.reshape(-1, 1)
    return (score, score > 0.5)


# speedup vs baseline: 5.1616x; 1.2400x over previous
import functools

import jax
import jax.numpy as jnp
from jax import lax
from jax.experimental import pallas as pl
from jax.experimental.pallas import tpu as pltpu
from jax.experimental.pallas import tpu_sc as plsc

_NUM_CORES = 2
_NUM_SUBCORES = 16
_NW = _NUM_CORES * _NUM_SUBCORES
_LANES = 16

_N = 10000
_E = 320000
_D = 128
_DT = 2 * _D

_C = 80
_PER_W = _E // _NW
_CHUNKS_W = _PER_W // _C


def _tables_kernel(h_ref, t_ref):
    hh = h_ref[...]
    t_ref[...] = jnp.concatenate([jnp.sin(hh), jnp.cos(hh)], axis=1)


def _make_table(h):
    rows = 1000
    grid = _N // rows
    return pl.pallas_call(
        _tables_kernel,
        grid=(grid,),
        in_specs=[pl.BlockSpec((rows, _D), lambda i: (i, 0))],
        out_specs=pl.BlockSpec((rows, _DT), lambda i: (i, 0)),
        out_shape=jax.ShapeDtypeStruct((_N, _DT), jnp.float32),
    )(h)


def _round_bf16(t):
    i = plsc.bitcast(t, jnp.int32)
    lsb = lax.shift_right_logical(i, 16) & 1
    i = (i + 0x7FFF + lsb) & jnp.int32(-65536)
    return plsc.bitcast(i, jnp.float32)


def _sc_kernel(t_hbm, src_hbm, dst_hbm, w_hbm, out_hbm,
               sidx, didx, arows0, brows0, arows1, brows1,
               scbuf, pbuf0, pbuf1, stage, wbuf,
               sem_a0, sem_b0, sem_a1, sem_b1, sem_o0, sem_o1):
    wid = lax.axis_index("s") * _NUM_CORES + lax.axis_index("c")
    base = wid * _PER_W
    iota = lax.iota(jnp.int32, _LANES)

    pltpu.sync_copy(w_hbm, wbuf)
    wv = [wbuf[pl.ds(j * _LANES, _LANES)] for j in range(_D // _LANES)]

    pltpu.sync_copy(src_hbm.at[pl.ds(base, _PER_W)], sidx)
    pltpu.sync_copy(dst_hbm.at[pl.ds(base, _PER_W)], didx)

    def fire(t, arows, brows, sa, sb):
        s_ref = sidx.at[pl.ds(t * _C, _C)]
        d_ref = didx.at[pl.ds(t * _C, _C)]
        pltpu.async_copy(t_hbm.at[s_ref], arows, sa)
        pltpu.async_copy(t_hbm.at[d_ref], brows, sb)

    def drain(t, arows, brows, sa, sb):
        s_ref = sidx.at[pl.ds(t * _C, _C)]
        d_ref = didx.at[pl.ds(t * _C, _C)]
        pltpu.make_async_copy(t_hbm.at[s_ref], arows, sa).wait()
        pltpu.make_async_copy(t_hbm.at[d_ref], brows, sb).wait()

    def compute(t, arows, brows, pbuf, so):
        off = base + t * _C

        def blk_body(blk, _):
            for u in range(_LANES):
                e = blk * _LANES + u
                acc = jnp.zeros((_LANES,), jnp.float32)
                for j in range(_D // _LANES):
                    sin_s = arows[e, pl.ds(j * _LANES, _LANES)]
                    cos_s = arows[e, pl.ds(_D + j * _LANES, _LANES)]
                    sin_o = brows[e, pl.ds(j * _LANES, _LANES)]
                    cos_o = brows[e, pl.ds(_D + j * _LANES, _LANES)]
                    sin_d = sin_s * cos_o - cos_s * sin_o
                    acc = acc + _round_bf16(sin_d) * wv[j]
                stage[pl.ds(u * _LANES, _LANES)] = acc
            score = jnp.zeros((_LANES,), jnp.float32)
            col = iota * _LANES
            for c in range(_LANES):
                score = score + plsc.load_gather(stage, [col + c])
            scbuf[pl.ds(blk * _LANES, _LANES)] = score
            return _

        lax.fori_loop(0, _C // _LANES, blk_body, None)

        for w0 in (0, 32, 48):
            idx_e = w0 + 2 * iota
            idx_o = idx_e + 1
            x0 = plsc.load_gather(scbuf, [idx_e])
            x1 = plsc.load_gather(scbuf, [idx_o])
            m = jnp.maximum(x0, x1)
            e0 = jnp.exp(x0 - m)
            e1 = jnp.exp(x1 - m)
            s = e0 + e1
            plsc.store_scatter(pbuf, [idx_e], e0 / s)
            plsc.store_scatter(pbuf, [idx_o], e1 / s)

        pltpu.async_copy(pbuf, out_hbm.at[pl.ds(off, _C)], so)

    def wait_out(t, pbuf, so):
        off = base + t * _C
        pltpu.make_async_copy(pbuf, out_hbm.at[pl.ds(off, _C)], so).wait()

    fire(0, arows0, brows0, sem_a0, sem_b0)

    def body(g, _):
        t0 = 2 * g
        fire(t0 + 1, arows1, brows1, sem_a1, sem_b1)
        drain(t0, arows0, brows0, sem_a0, sem_b0)
        @pl.when(g > 0)
        def _():
            wait_out(t0 - 2, pbuf0, sem_o0)
        compute(t0, arows0, brows0, pbuf0, sem_o0)
        fire(t0 + 2, arows0, brows0, sem_a0, sem_b0)
        drain(t0 + 1, arows1, brows1, sem_a1, sem_b1)
        @pl.when(g > 0)
        def _():
            wait_out(t0 - 1, pbuf1, sem_o1)
        compute(t0 + 1, arows1, brows1, pbuf1, sem_o1)
        return _

    lax.fori_loop(0, (_CHUNKS_W - 1) // 2, body, None)
    drain(_CHUNKS_W - 1, arows0, brows0, sem_a0, sem_b0)
    wait_out(_CHUNKS_W - 3, pbuf0, sem_o0)
    compute(_CHUNKS_W - 1, arows0, brows0, pbuf0, sem_o0)
    wait_out(_CHUNKS_W - 2, pbuf1, sem_o1)
    wait_out(_CHUNKS_W - 1, pbuf0, sem_o0)


@functools.partial(
    pl.kernel,
    mesh=plsc.VectorSubcoreMesh(core_axis_name="c", subcore_axis_name="s"),
    compiler_params=pltpu.CompilerParams(needs_layout_passes=False),
    out_type=jax.ShapeDtypeStruct((_E,), jnp.float32),
    scratch_types=[
        pltpu.VMEM((_PER_W,), jnp.int32),
        pltpu.VMEM((_PER_W,), jnp.int32),
        pltpu.VMEM((_C, _DT), jnp.float32),
        pltpu.VMEM((_C, _DT), jnp.float32),
        pltpu.VMEM((_C, _DT), jnp.float32),
        pltpu.VMEM((_C, _DT), jnp.float32),
        pltpu.VMEM((_C,), jnp.float32),
        pltpu.VMEM((_C,), jnp.float32),
        pltpu.VMEM((_C,), jnp.float32),
        pltpu.VMEM((_LANES * _LANES,), jnp.float32),
        pltpu.VMEM((_D,), jnp.float32),
        pltpu.SemaphoreType.DMA,
        pltpu.SemaphoreType.DMA,
        pltpu.SemaphoreType.DMA,
        pltpu.SemaphoreType.DMA,
        pltpu.SemaphoreType.DMA,
        pltpu.SemaphoreType.DMA,
    ],
)
def _sc_scores(t_hbm, src_hbm, dst_hbm, w_hbm, out_hbm,
               sidx, didx, arows0, brows0, arows1, brows1,
               scbuf, pbuf0, pbuf1, stage, wbuf,
               sem_a0, sem_b0, sem_a1, sem_b1, sem_o0, sem_o1):
    _sc_kernel(t_hbm, src_hbm, dst_hbm, w_hbm, out_hbm,
               sidx, didx, arows0, brows0, arows1, brows1,
               scbuf, pbuf0, pbuf1, stage, wbuf,
               sem_a0, sem_b0, sem_a1, sem_b1, sem_o0, sem_o1)


def kernel(h, edge_index, W, b):
    del b
    T = _make_table(h)
    wi = lax.bitcast_convert_type(W.reshape(-1), jnp.int32)
    wlsb = lax.shift_right_logical(wi, 16) & 1
    wi = (wi + 0x7FFF + wlsb) & jnp.int32(-65536)
    w_bf = lax.bitcast_convert_type(wi, jnp.float32)
    src = edge_index[0]
    dst = edge_index[1]
    probs = _sc_scores(T, src, dst, w_bf)
    score = probs.reshape(-1, 1)
    return (score, score > 0.5)
